# baseline (speedup 1.0000x reference)
import functools

import jax
import jax.numpy as jnp
from jax import lax
from jax.experimental import pallas as pl
from jax.experimental.pallas import tpu as pltpu
from jax.experimental.pallas import tpu_sc as plsc

Q = 1024
D = 128
N = 100000
CH = 128
TB = 2048
QB = 256
NPAD = 100352
NJ = NPAD // TB
NCH = NPAD // CH
NCHP = 896
KCH = 16
CAND = KCH * CH
NN = 15
NCLS = 10

_BIG_F = 3.0e38
_BIG_I = 1 << 30


def _dist_body(x_ref, tx_ref, xsq_ref, tsq_ref, d2_ref, cm_ref):
    xb = x_ref[...]
    tb = tx_ref[...]
    mm = lax.dot_general(xb, tb, (((1,), (1,)), ((), ())),
                         preferred_element_type=jnp.float32)
    tsq = tsq_ref[0, 0, :][None, :]
    xsq = xsq_ref[:, 0:1]
    d2 = (xsq + tsq) - 2.0 * mm
    gcol = lax.broadcasted_iota(jnp.int32, (Q, TB), 1) + pl.program_id(0) * TB
    d2 = jnp.where(gcol < N, d2, 1e9)
    d2_ref[...] = d2.reshape(Q, TB // CH, CH)
    lane = lax.broadcasted_iota(jnp.int32, (Q, TB // CH), 1)
    cm = jnp.zeros((Q, TB // CH), jnp.float32)
    for c in range(TB // CH):
        mins = jnp.min(d2[:, c * CH:(c + 1) * CH], axis=1)
        cm = cm + jnp.where(lane == c, mins[:, None], 0.0)
    cm_ref[...] = cm.reshape(1, Q, TB // CH)


def _distances(x, tx_pad, xsq_t, tsq_r):
    return pl.pallas_call(
        _dist_body,
        grid=(NJ,),
        in_specs=[
            pl.BlockSpec((Q, D), lambda j: (0, 0)),
            pl.BlockSpec((TB, D), lambda j: (j, 0)),
            pl.BlockSpec((Q, 128), lambda j: (0, 0)),
            pl.BlockSpec((1, 1, TB), lambda j: (j, 0, 0)),
        ],
        out_specs=[
            pl.BlockSpec((Q, TB // CH, CH), lambda j: (0, j, 0)),
            pl.BlockSpec((1, Q, TB // CH), lambda j: (j, 0, 0)),
        ],
        out_shape=[
            jax.ShapeDtypeStruct((Q, NCH, CH), jnp.float32),
            jax.ShapeDtypeStruct((NJ, Q, TB // CH), jnp.float32),
        ],
    )(x, tx_pad, xsq_t, tsq_r)


def _select_body(cm_ref, ids_ref, fidx_ref):
    w = cm_ref[...]
    cpb = TB // CH
    jio = lax.broadcasted_iota(jnp.int32, (NJ, QB, cpb), 0)
    cio = lax.broadcasted_iota(jnp.int32, (NJ, QB, cpb), 2)
    enc = jio * cpb + cio
    lane = lax.broadcasted_iota(jnp.int32, (QB, 128), 1)
    ids = jnp.zeros((QB, 128), jnp.int32)
    for i in range(KCH):
        m = jnp.min(jnp.min(w, axis=0), axis=1, keepdims=True)
        cand = jnp.where(w == m[None, :, :], enc, _BIG_I)
        first = jnp.min(jnp.min(cand, axis=0), axis=1, keepdims=True)
        ids = ids + jnp.where(lane == i, first, 0)
        w = jnp.where(enc == first[None, :, :], _BIG_F, w)
    s = jnp.where(lane < KCH, ids, _BIG_I)
    srt = jnp.zeros((QB, 128), jnp.int32)
    for j in range(KCH):
        mn = jnp.min(s, axis=1, keepdims=True)
        srt = srt + jnp.where(lane == j, mn, 0)
        s = jnp.where(s == mn, _BIG_I, s)
    qrow = (lax.broadcasted_iota(jnp.int32, (QB, 128), 0)
            + pl.program_id(0) * QB)
    valid = lane < KCH
    ids_ref[...] = jnp.where(valid, srt, 0)
    fidx_ref[...] = jnp.where(valid, srt + qrow * NCH, 0)


def _select_chunks(cm3):
    return pl.pallas_call(
        _select_body,
        grid=(Q // QB,),
        in_specs=[pl.BlockSpec((NJ, QB, TB // CH), lambda q: (0, q, 0))],
        out_specs=[
            pl.BlockSpec((QB, 128), lambda q: (q, 0)),
            pl.BlockSpec((QB, 128), lambda q: (q, 0)),
        ],
        out_shape=[
            jax.ShapeDtypeStruct((Q, 128), jnp.int32),
            jax.ShapeDtypeStruct((Q, 128), jnp.int32),
        ],
    )(cm3)


_NC = 2
_NS = 16
_NW = _NC * _NS
_ROWS = Q * KCH
_RPW = _ROWS // _NW
_SUB = 128


def _gather_body(d2_tab, y_tab, idx_d2, idx_lab,
                 out_d2, out_lab, idxf, idxl, rows_f, rows_i,
                 gf, gi, sf, si):
    wid = lax.axis_index("s") * _NC + lax.axis_index("c")
    nb = _RPW // _SUB
    for b in range(nb):
        base = wid * _RPW + b * _SUB
        pltpu.sync_copy(idx_d2.at[pl.ds(base, _SUB)], idxf.at[b])
        pltpu.sync_copy(idx_lab.at[pl.ds(base, _SUB)], idxl.at[b])
    for b in range(nb):
        base = wid * _RPW + b * _SUB
        hf = pltpu.async_copy(d2_tab.at[idxf.at[b]], rows_f, gf)
        hi = pltpu.async_copy(y_tab.at[idxl.at[b]], rows_i, gi)
        hf.wait()
        hsf = pltpu.async_copy(rows_f, out_d2.at[pl.ds(base, _SUB)], sf)
        hi.wait()
        hsi = pltpu.async_copy(rows_i, out_lab.at[pl.ds(base, _SUB)], si)
        hsf.wait()
        hsi.wait()


def _gather_candidates(d2_tab, y_tab, idx_d2, idx_lab):
    mesh = plsc.VectorSubcoreMesh(core_axis_name="c", subcore_axis_name="s")
    f = functools.partial(
        pl.kernel,
        mesh=mesh,
        out_type=[
            jax.ShapeDtypeStruct((_ROWS, CH), jnp.float32),
            jax.ShapeDtypeStruct((_ROWS, CH), jnp.int32),
        ],
        scratch_types=[
            pltpu.VMEM((_RPW // _SUB, _SUB), jnp.int32),
            pltpu.VMEM((_RPW // _SUB, _SUB), jnp.int32),
            pltpu.VMEM((_SUB, CH), jnp.float32),
            pltpu.VMEM((_SUB, CH), jnp.int32),
            pltpu.SemaphoreType.DMA,
            pltpu.SemaphoreType.DMA,
            pltpu.SemaphoreType.DMA,
            pltpu.SemaphoreType.DMA,
        ],
    )(_gather_body)
    return f(d2_tab, y_tab, idx_d2, idx_lab)


def _vote_body(d2c_ref, lab_ref, preds_ref, probs_ref):
    v = d2c_ref[...]
    labs = lab_ref[...]
    lane = lax.broadcasted_iota(jnp.int32, (QB, CAND), 1)
    cls = lax.broadcasted_iota(jnp.int32, (QB, 128), 1)
    votes = jnp.zeros((QB, 128), jnp.float32)
    for _ in range(NN):
        m = jnp.min(v, axis=1, keepdims=True)
        pos = jnp.min(jnp.where(v == m, lane, _BIG_I), axis=1, keepdims=True)
        sel = lane == pos
        labsel = jnp.sum(jnp.where(sel, labs, 0), axis=1, keepdims=True)
        votes = votes + jnp.where(cls == labsel, 1.0, 0.0)
        v = jnp.where(sel, _BIG_F, v)
    probs_ref[...] = votes / float(NN)
    pv = jnp.where(cls < NCLS, votes, -1.0)
    mx = jnp.max(pv, axis=1, keepdims=True)
    pred = jnp.min(jnp.where(pv == mx, cls, _BIG_I), axis=1, keepdims=True)
    preds_ref[...] = jnp.broadcast_to(pred, (QB, 128))


def _vote(cand_d2, cand_lab):
    return pl.pallas_call(
        _vote_body,
        grid=(Q // QB,),
        in_specs=[
            pl.BlockSpec((QB, CAND), lambda q: (q, 0)),
            pl.BlockSpec((QB, CAND), lambda q: (q, 0)),
        ],
        out_specs=[
            pl.BlockSpec((QB, 128), lambda q: (q, 0)),
            pl.BlockSpec((QB, 128), lambda q: (q, 0)),
        ],
        out_shape=[
            jax.ShapeDtypeStruct((Q, 128), jnp.int32),
            jax.ShapeDtypeStruct((Q, 128), jnp.float32),
        ],
    )(cand_d2, cand_lab)


def kernel(x, train_x, train_y):
    x_sq = jnp.sum(x * x, axis=1, keepdims=True)
    t_sq = jnp.sum(train_x * train_x, axis=1)
    xsq_t = jnp.broadcast_to(x_sq, (Q, 128))
    t_sq_pad = jnp.concatenate(
        [t_sq, jnp.full((NPAD - N,), 1e9, jnp.float32)]).reshape(NJ, 1, TB)
    ty_pad = jnp.concatenate(
        [train_y, jnp.zeros((NPAD - N,), jnp.int32)]).reshape(NCH, CH)

    d2, cm3 = _distances(x, train_x, xsq_t, t_sq_pad)
    ids_pad, fidx_pad = _select_chunks(cm3)
    idx_d2 = fidx_pad[:, :KCH].reshape(_ROWS)
    idx_lab = ids_pad[:, :KCH].reshape(_ROWS)

    cand_d2, cand_lab = _gather_candidates(
        d2.reshape(Q * NCH, CH), ty_pad, idx_d2, idx_lab)

    preds_pad, probs_pad = _vote(
        cand_d2.reshape(Q, CAND), cand_lab.reshape(Q, CAND))
    return preds_pad[:, 0], probs_pad[:, :NCLS]


# speedup vs baseline: 1.4259x; 1.4259x over previous
import functools

import jax
import jax.numpy as jnp
from jax import lax
from jax.experimental import pallas as pl
from jax.experimental.pallas import tpu as pltpu
from jax.experimental.pallas import tpu_sc as plsc

Q = 1024
D = 128
N = 100000
CH = 128
TB = 2048
QB = 256
NPAD = 100352
NJ = NPAD // TB
NCH = NPAD // CH
NCHP = 896
KCH = 16
CAND = KCH * CH
NN = 15
NCLS = 10

_BIG_F = 3.0e38
_BIG_I = 1 << 30


def _dist_body(x_ref, tx_ref, xsq_ref, tsq_ref, d2_ref, cm_ref):
    xb = x_ref[...]
    tb = tx_ref[...]
    mm = lax.dot_general(xb, tb, (((1,), (1,)), ((), ())),
                         preferred_element_type=jnp.float32)
    tsq = tsq_ref[0, 0, :][None, :]
    xsq = xsq_ref[:, 0:1]
    d2 = (xsq + tsq) - 2.0 * mm
    gcol = lax.broadcasted_iota(jnp.int32, (Q, TB), 1) + pl.program_id(0) * TB
    d2 = jnp.where(gcol < N, d2, 1e9)
    d2_ref[...] = d2.reshape(Q, TB // CH, CH)
    sub = lax.broadcasted_iota(jnp.int32, (TB // CH, Q), 0)
    cm = jnp.zeros((TB // CH, Q), jnp.float32)
    for c in range(TB // CH):
        mins = jnp.min(d2[:, c * CH:(c + 1) * CH], axis=1)
        cm = cm + jnp.where(sub == c, mins[None, :], 0.0)
    cm_ref[...] = cm


def _distances(x, tx_pad, xsq_t, tsq_r):
    return pl.pallas_call(
        _dist_body,
        grid=(NJ,),
        in_specs=[
            pl.BlockSpec((Q, D), lambda j: (0, 0)),
            pl.BlockSpec((TB, D), lambda j: (j, 0)),
            pl.BlockSpec((Q, 128), lambda j: (0, 0)),
            pl.BlockSpec((1, 1, TB), lambda j: (j, 0, 0)),
        ],
        out_specs=[
            pl.BlockSpec((Q, TB // CH, CH), lambda j: (0, j, 0)),
            pl.BlockSpec((TB // CH, Q), lambda j: (j, 0)),
        ],
        out_shape=[
            jax.ShapeDtypeStruct((Q, NCH, CH), jnp.float32),
            jax.ShapeDtypeStruct((NCH, Q), jnp.float32),
        ],
    )(x, tx_pad, xsq_t, tsq_r)


def _select_body(cm_ref, ids_ref, fidx_ref):
    w = cm_ref[...]
    row = lax.broadcasted_iota(jnp.int32, (NCH, QB), 0)
    sub = lax.broadcasted_iota(jnp.int32, (KCH, QB), 0)
    ids = jnp.zeros((KCH, QB), jnp.int32)
    for i in range(KCH):
        m = jnp.min(w, axis=0, keepdims=True)
        first = jnp.min(jnp.where(w == m, row, _BIG_I), axis=0,
                        keepdims=True)
        ids = ids + jnp.where(sub == i, first, 0)
        w = jnp.where(row == first, _BIG_F, w)
    s = ids
    srt = jnp.zeros((KCH, QB), jnp.int32)
    for j in range(KCH):
        mn = jnp.min(s, axis=0, keepdims=True)
        srt = srt + jnp.where(sub == j, mn, 0)
        s = jnp.where(s == mn, _BIG_I, s)
    qcol = (lax.broadcasted_iota(jnp.int32, (KCH, QB), 1)
            + pl.program_id(0) * QB)
    ids_ref[...] = srt
    fidx_ref[...] = srt + qcol * NCH


def _select_chunks(cmt):
    return pl.pallas_call(
        _select_body,
        grid=(Q // QB,),
        in_specs=[pl.BlockSpec((NCH, QB), lambda q: (0, q))],
        out_specs=[
            pl.BlockSpec((KCH, QB), lambda q: (0, q)),
            pl.BlockSpec((KCH, QB), lambda q: (0, q)),
        ],
        out_shape=[
            jax.ShapeDtypeStruct((KCH, Q), jnp.int32),
            jax.ShapeDtypeStruct((KCH, Q), jnp.int32),
        ],
    )(cmt)


_NC = 2
_NS = 16
_NW = _NC * _NS
_ROWS = Q * KCH
_RPW = _ROWS // _NW
_SUB = 128


def _gather_body(d2_tab, y_tab, idx_d2, idx_lab,
                 out_d2, out_lab, idxf, idxl, rows_f, rows_i,
                 gf, gi, sf, si):
    wid = lax.axis_index("s") * _NC + lax.axis_index("c")
    nb = _RPW // _SUB
    for b in range(nb):
        base = wid * _RPW + b * _SUB
        pltpu.sync_copy(idx_d2.at[pl.ds(base, _SUB)], idxf.at[b])
        pltpu.sync_copy(idx_lab.at[pl.ds(base, _SUB)], idxl.at[b])
    for b in range(nb):
        base = wid * _RPW + b * _SUB
        hf = pltpu.async_copy(d2_tab.at[idxf.at[b]], rows_f, gf)
        hi = pltpu.async_copy(y_tab.at[idxl.at[b]], rows_i, gi)
        hf.wait()
        hsf = pltpu.async_copy(rows_f, out_d2.at[pl.ds(base, _SUB)], sf)
        hi.wait()
        hsi = pltpu.async_copy(rows_i, out_lab.at[pl.ds(base, _SUB)], si)
        hsf.wait()
        hsi.wait()


def _gather_candidates(d2_tab, y_tab, idx_d2, idx_lab):
    mesh = plsc.VectorSubcoreMesh(core_axis_name="c", subcore_axis_name="s")
    f = functools.partial(
        pl.kernel,
        mesh=mesh,
        out_type=[
            jax.ShapeDtypeStruct((_ROWS, CH), jnp.float32),
            jax.ShapeDtypeStruct((_ROWS, CH), jnp.int32),
        ],
        scratch_types=[
            pltpu.VMEM((_RPW // _SUB, _SUB), jnp.int32),
            pltpu.VMEM((_RPW // _SUB, _SUB), jnp.int32),
            pltpu.VMEM((_SUB, CH), jnp.float32),
            pltpu.VMEM((_SUB, CH), jnp.int32),
            pltpu.SemaphoreType.DMA,
            pltpu.SemaphoreType.DMA,
            pltpu.SemaphoreType.DMA,
            pltpu.SemaphoreType.DMA,
        ],
    )(_gather_body)
    return f(d2_tab, y_tab, idx_d2, idx_lab)


def _vote_body(d2c_ref, lab_ref, preds_ref, probs_ref):
    v = d2c_ref[...]
    labs = lab_ref[...]
    lane = lax.broadcasted_iota(jnp.int32, (QB, CAND), 1)
    cls = lax.broadcasted_iota(jnp.int32, (QB, 128), 1)
    votes = jnp.zeros((QB, 128), jnp.float32)
    for _ in range(NN):
        m = jnp.min(v, axis=1, keepdims=True)
        pos = jnp.min(jnp.where(v == m, lane, _BIG_I), axis=1, keepdims=True)
        sel = lane == pos
        labsel = jnp.sum(jnp.where(sel, labs, 0), axis=1, keepdims=True)
        votes = votes + jnp.where(cls == labsel, 1.0, 0.0)
        v = jnp.where(sel, _BIG_F, v)
    probs_ref[...] = votes / float(NN)
    pv = jnp.where(cls < NCLS, votes, -1.0)
    mx = jnp.max(pv, axis=1, keepdims=True)
    pred = jnp.min(jnp.where(pv == mx, cls, _BIG_I), axis=1, keepdims=True)
    preds_ref[...] = jnp.broadcast_to(pred, (QB, 128))


def _vote(cand_d2, cand_lab):
    return pl.pallas_call(
        _vote_body,
        grid=(Q // QB,),
        in_specs=[
            pl.BlockSpec((QB, CAND), lambda q: (q, 0)),
            pl.BlockSpec((QB, CAND), lambda q: (q, 0)),
        ],
        out_specs=[
            pl.BlockSpec((QB, 128), lambda q: (q, 0)),
            pl.BlockSpec((QB, 128), lambda q: (q, 0)),
        ],
        out_shape=[
            jax.ShapeDtypeStruct((Q, 128), jnp.int32),
            jax.ShapeDtypeStruct((Q, 128), jnp.float32),
        ],
    )(cand_d2, cand_lab)


def kernel(x, train_x, train_y):
    x_sq = jnp.sum(x * x, axis=1, keepdims=True)
    t_sq = jnp.sum(train_x * train_x, axis=1)
    xsq_t = jnp.broadcast_to(x_sq, (Q, 128))
    t_sq_pad = jnp.concatenate(
        [t_sq, jnp.full((NPAD - N,), 1e9, jnp.float32)]).reshape(NJ, 1, TB)
    ty_pad = jnp.concatenate(
        [train_y, jnp.zeros((NPAD - N,), jnp.int32)]).reshape(NCH, CH)

    d2, cmt = _distances(x, train_x, xsq_t, t_sq_pad)
    ids16, fidx16 = _select_chunks(cmt)
    idx_d2 = fidx16.T.reshape(_ROWS)
    idx_lab = ids16.T.reshape(_ROWS)

    cand_d2, cand_lab = _gather_candidates(
        d2.reshape(Q * NCH, CH), ty_pad, idx_d2, idx_lab)

    preds_pad, probs_pad = _vote(
        cand_d2.reshape(Q, CAND), cand_lab.reshape(Q, CAND))
    return preds_pad[:, 0], probs_pad[:, :NCLS]
